# baseline (speedup 1.0000x reference)
import functools

import jax
import jax.numpy as jnp
from jax import lax
from jax.experimental import pallas as pl
from jax.experimental.pallas import tpu as pltpu
from jax.experimental.pallas import tpu_sc as plsc

N = 10000
NPAD = 10240
E = 320000
EPAD = 327680
WIN = 128
NSUB = 16
NCORE = 2
ROWS_PER_SUB = NPAD // NSUB

_mesh = plsc.VectorSubcoreMesh(core_axis_name="c", subcore_axis_name="s")


def _deg_body(dst_hbm, zeros_hbm, ones_hbm, out_hbm, deg_sp, idx_v, ones_v, sem):
    c = lax.axis_index("c")
    s = lax.axis_index("s")
    pltpu.sync_copy(zeros_hbm.at[pl.ds(s * ROWS_PER_SUB, ROWS_PER_SUB)],
                    deg_sp.at[pl.ds(s * ROWS_PER_SUB, ROWS_PER_SUB)])
    pltpu.sync_copy(ones_hbm, ones_v)
    plsc.subcore_barrier()
    per_worker = EPAD // (NCORE * NSUB)
    base = (c * NSUB + s) * per_worker

    def win(j, _):
        pltpu.sync_copy(dst_hbm.at[pl.ds(base + j * WIN, WIN)], idx_v)
        pltpu.sync_copy(ones_v, deg_sp.at[idx_v], add=True)
        return 0

    lax.fori_loop(0, per_worker // WIN, win, 0)
    plsc.subcore_barrier()
    pltpu.sync_copy(deg_sp.at[pl.ds(s * ROWS_PER_SUB, ROWS_PER_SUB)],
                    out_hbm.at[c].at[pl.ds(s * ROWS_PER_SUB, ROWS_PER_SUB)])


_deg_kernel = pl.kernel(
    _deg_body,
    out_type=jax.ShapeDtypeStruct((NCORE, NPAD), jnp.float32),
    mesh=_mesh,
    scratch_types=[
        pltpu.VMEM_SHARED((NPAD,), jnp.float32),
        pltpu.VMEM((WIN,), jnp.int32),
        pltpu.VMEM((WIN,), jnp.float32),
        pltpu.SemaphoreType.DMA,
    ],
)


NBUF = 2
NWROW = EPAD // WIN


def _agg_pipeline(gather_ref, acc_sp, sd_hbm, idxb, rows,
                  isem, gsem, ssem, row_base, nwin):
    def idx_fire(w0, b):
        pltpu.async_copy(sd_hbm.at[w0 + b], idxb[b], isem[b])

    def idx_wait(w0, b):
        pltpu.make_async_copy(sd_hbm.at[w0 + b], idxb[b], isem[b]).wait()

    def gather_fire(b):
        pltpu.async_copy(gather_ref.at[idxb[b].at[0]], rows[b], gsem[b])

    def gather_wait(b):
        pltpu.make_async_copy(gather_ref.at[idxb[b].at[0]], rows[b],
                              gsem[b]).wait()

    def scat_fire(b):
        pltpu.async_copy(rows[b], acc_sp.at[idxb[b].at[1]], ssem[b], add=True)

    def scat_wait(b):
        pltpu.make_async_copy(rows[b], acc_sp.at[idxb[b].at[1]],
                              ssem[b]).wait()

    for b in range(NBUF):
        idx_fire(row_base, b)
    for b in range(NBUF):
        idx_wait(row_base, b)
        gather_fire(b)
    for b in range(NBUF):
        gather_wait(b)
        scat_fire(b)

    def macro(j, _):
        w0 = row_base + j * NBUF
        for b in range(NBUF):
            scat_wait(b)
            idx_fire(w0, b)
        for b in range(NBUF):
            idx_wait(w0, b)
            gather_fire(b)
        for b in range(NBUF):
            gather_wait(b)
            scat_fire(b)
        return 0

    lax.fori_loop(1, nwin // NBUF, macro, 0)
    for b in range(NBUF):
        scat_wait(b)


def _agg_body(hs_hbm, sd_hbm, out_hbm, acc_sp,
              i0, i1, r0, r1, q0, q1, g0, g1, s0, s1, *, esplit):
    c = lax.axis_index("c")
    s = lax.axis_index("s")
    pltpu.sync_copy(hs_hbm.at[c].at[pl.ds(s * ROWS_PER_SUB, ROWS_PER_SUB)],
                    acc_sp.at[pl.ds(s * ROWS_PER_SUB, ROWS_PER_SUB)])
    plsc.subcore_barrier()
    if esplit:
        w = c * NSUB + s
        row_base = w * (NWROW // (NCORE * NSUB))
        nwin = NWROW // (NCORE * NSUB)
        gref = hs_hbm.at[0]
    else:
        row_base = s * (NWROW // NSUB)
        nwin = NWROW // NSUB
        gref = hs_hbm.at[c]
    _agg_pipeline(gref, acc_sp, sd_hbm,
                  (i0, i1), (r0, r1),
                  (q0, q1), (g0, g1), (s0, s1),
                  row_base, nwin)
    plsc.subcore_barrier()
    pltpu.sync_copy(acc_sp.at[pl.ds(s * ROWS_PER_SUB, ROWS_PER_SUB)],
                    out_hbm.at[c].at[pl.ds(s * ROWS_PER_SUB, ROWS_PER_SUB)])


def _make_agg(esplit):
    return pl.kernel(
        functools.partial(_agg_body, esplit=esplit),
        out_type=jax.ShapeDtypeStruct((NCORE, NPAD, 128), jnp.float32),
        mesh=_mesh,
        scratch_types=[
            pltpu.VMEM_SHARED((NPAD, 128), jnp.float32),
            pltpu.VMEM((2, WIN), jnp.int32),
            pltpu.VMEM((2, WIN), jnp.int32),
            pltpu.VMEM((WIN, 128), jnp.float32),
            pltpu.VMEM((WIN, 128), jnp.float32),
            pltpu.SemaphoreType.DMA,
            pltpu.SemaphoreType.DMA,
            pltpu.SemaphoreType.DMA,
            pltpu.SemaphoreType.DMA,
            pltpu.SemaphoreType.DMA,
            pltpu.SemaphoreType.DMA,
        ],
    )


_agg128 = _make_agg(False)
_agg_esplit = _make_agg(True)


HALF = NPAD // 2
RPS2 = HALF // NSUB
WIN2 = 128
NWROW2 = EPAD // WIN2


def _agg256_body(hs_hbm, sdp_hbm, par_hbm, out_hbm, acc_sp, par_sm,
                 i0, i1, r0, q0, q1, g0, t0):
    c = lax.axis_index("c")
    s = lax.axis_index("s")
    pltpu.sync_copy(hs_hbm.at[pl.ds(c * HALF + s * RPS2, RPS2)],
                    acc_sp.at[pl.ds(s * RPS2, RPS2)])
    pltpu.sync_copy(par_hbm, par_sm)
    prow = par_sm[c]
    begin = prow[0]
    end = prow[1]
    pw = prow[2]
    plsc.subcore_barrier()
    base = begin + s * pw
    idxb = (i0, i1)
    isem = (q0, q1)
    sd_c = sdp_hbm.at[c]

    def widx(j):
        return jnp.where(base + j < end, base + j, NWROW2)

    def idx_fire(j, b):
        pltpu.async_copy(sd_c.at[widx(j)], idxb[b], isem[b])

    def idx_wait(j, b):
        pltpu.make_async_copy(sd_c.at[widx(j)], idxb[b], isem[b]).wait()

    idx_fire(0, 0)

    def step(st, _):
        for b in range(2):
            j = st * 2 + b
            idx_wait(j, b)
            pltpu.async_copy(hs_hbm.at[idxb[b].at[0]], r0, g0)
            idx_fire(j + 1, 1 - b)
            pltpu.make_async_copy(hs_hbm.at[idxb[b].at[0]], r0, g0).wait()
            pltpu.async_copy(r0, acc_sp.at[idxb[b].at[1]], t0, add=True)
            pltpu.make_async_copy(r0, acc_sp.at[idxb[b].at[1]], t0).wait()
        return 0

    lax.fori_loop(0, pw // 2, step, 0)
    idx_wait(pw, 0)
    plsc.subcore_barrier()
    pltpu.sync_copy(acc_sp.at[pl.ds(s * RPS2, RPS2)],
                    out_hbm.at[c].at[pl.ds(s * RPS2, RPS2)])


_agg256 = pl.kernel(
    _agg256_body,
    out_type=jax.ShapeDtypeStruct((NCORE, HALF, 2, 128), jnp.float32),
    mesh=_mesh,
    scratch_types=[
        pltpu.VMEM_SHARED((HALF, 2, 128), jnp.float32),
        pltpu.VMEM((NCORE, 3), jnp.int32),
        pltpu.VMEM((2, WIN2), jnp.int32),
        pltpu.VMEM((2, WIN2), jnp.int32),
        pltpu.VMEM((WIN2, 2, 128), jnp.float32),
        pltpu.SemaphoreType.DMA,
        pltpu.SemaphoreType.DMA,
        pltpu.SemaphoreType.DMA,
        pltpu.SemaphoreType.DMA,
    ],
)


BM = 2048


def _l0_body(x_ref, w_ref, dis_ref, out_ref):
    h = jnp.dot(x_ref[...], w_ref[...], preferred_element_type=jnp.float32)
    out_ref[...] = h * dis_ref[...]


def _mid_body(agg_ref, dis_ref, b_ref, w_ref, out_ref, *, last):
    t = agg_ref[...]
    y = jnp.maximum(t * dis_ref[...] + b_ref[...], 0.0)
    h = jnp.dot(y, w_ref[...], preferred_element_type=jnp.float32)
    h = h * dis_ref[...]
    if last:
        out_ref[0] = h
        out_ref[1] = jnp.zeros_like(h)
    else:
        out_ref[...] = h


def _final_body(agg_ref, dis_ref, b_ref, out_ref):
    t = agg_ref[0] + agg_ref[1]
    y = t * dis_ref[...] + b_ref[...]
    col = lax.broadcasted_iota(jnp.int32, y.shape, 1)
    ym = jnp.where(col < 40, y, -jnp.inf)
    m = jnp.max(ym, axis=1, keepdims=True)
    lse = jnp.log(jnp.sum(jnp.exp(ym - m), axis=1, keepdims=True)) + m
    out_ref[...] = y - lse


def _tc_l0(x, w0, dis):
    grid = (NPAD // BM,)
    return pl.pallas_call(
        _l0_body,
        grid=grid,
        in_specs=[
            pl.BlockSpec((BM, 128), lambda i: (i, 0)),
            pl.BlockSpec((128, 256), lambda i: (0, 0)),
            pl.BlockSpec((BM, 1), lambda i: (i, 0)),
        ],
        out_specs=pl.BlockSpec((BM, 256), lambda i: (i, 0)),
        out_shape=jax.ShapeDtypeStruct((NPAD, 256), jnp.float32),
    )(x, w0, dis)


def _tc_mid(agg, dis, b, w, last):
    grid = (NPAD // BM,)
    kout = w.shape[1]
    if last:
        out_specs = pl.BlockSpec((NCORE, BM, kout), lambda i: (0, i, 0))
        out_shape = jax.ShapeDtypeStruct((NCORE, NPAD, kout), jnp.float32)
    else:
        out_specs = pl.BlockSpec((BM, kout), lambda i: (i, 0))
        out_shape = jax.ShapeDtypeStruct((NPAD, kout), jnp.float32)
    return pl.pallas_call(
        functools.partial(_mid_body, last=last),
        grid=grid,
        in_specs=[
            pl.BlockSpec((BM, 256), lambda i: (i, 0)),
            pl.BlockSpec((BM, 1), lambda i: (i, 0)),
            pl.BlockSpec((1, 256), lambda i: (0, 0)),
            pl.BlockSpec((256, kout), lambda i: (0, 0)),
        ],
        out_specs=out_specs,
        out_shape=out_shape,
    )(agg, dis, b, w)


def _tc_final(agg, dis, b):
    grid = (NPAD // BM,)
    return pl.pallas_call(
        _final_body,
        grid=grid,
        in_specs=[
            pl.BlockSpec((NCORE, BM, 128), lambda i: (0, i, 0)),
            pl.BlockSpec((BM, 1), lambda i: (i, 0)),
            pl.BlockSpec((1, 128), lambda i: (0, 0)),
        ],
        out_specs=pl.BlockSpec((BM, 128), lambda i: (i, 0)),
        out_shape=jax.ShapeDtypeStruct((NPAD, 128), jnp.float32),
    )(agg, dis, b)


def kernel(x, edge_index, W0, b0, W1, b1, W2, b2, W3, b3):
    src = edge_index[0].astype(jnp.int32)
    dst = edge_index[1].astype(jnp.int32)
    pad = N + (jnp.arange(EPAD - E, dtype=jnp.int32) % (NPAD - N))
    src_p = jnp.concatenate([src, pad])
    dst_p = jnp.concatenate([dst, pad])
    dst_flat = dst_p
    sd = jnp.stack([src_p.reshape(NWROW, WIN),
                    dst_p.reshape(NWROW, WIN)], axis=1)

    key = (dst_p >= HALF).astype(jnp.int32)
    c0 = EPAD - jnp.sum(key)
    pos = jnp.where(key == 0, jnp.cumsum(1 - key) - 1,
                    c0 + jnp.cumsum(key) - 1)
    ssrc = jnp.zeros((EPAD,), jnp.int32).at[pos].set(src_p)
    sdst = jnp.zeros((EPAD,), jnp.int32).at[pos].set(dst_p)
    iota_e = jnp.arange(EPAD, dtype=jnp.int32)
    dummy_src = N + (iota_e % (NPAD - N))
    dummy_dst = (iota_e * 37) % HALF
    keep0 = sdst < HALF
    s0a = jnp.where(keep0, ssrc, dummy_src)
    d0a = jnp.where(keep0, sdst, dummy_dst)
    s1a = jnp.where(keep0, dummy_src, ssrc)
    d1a = jnp.where(keep0, dummy_dst, sdst - HALF)
    dw_src = (N + (jnp.arange(WIN2, dtype=jnp.int32) % (NPAD - N)))[None, :]
    dw_dst = ((jnp.arange(WIN2, dtype=jnp.int32) * 37) % HALF)[None, :]
    sd0 = jnp.stack([jnp.concatenate([s0a.reshape(NWROW2, WIN2), dw_src]),
                     jnp.concatenate([d0a.reshape(NWROW2, WIN2), dw_dst])],
                    axis=1)
    sd1 = jnp.stack([jnp.concatenate([s1a.reshape(NWROW2, WIN2), dw_src]),
                     jnp.concatenate([d1a.reshape(NWROW2, WIN2), dw_dst])],
                    axis=1)
    sdp = jnp.stack([sd0, sd1])
    nw0 = (c0 + WIN2 - 1) // WIN2
    begin1 = c0 // WIN2
    nw1 = NWROW2 - begin1
    gran = NSUB * NBUF

    def _pw(nw):
        return ((nw + gran - 1) // gran) * NBUF

    par = jnp.stack([
        jnp.stack([jnp.int32(0), nw0.astype(jnp.int32), _pw(nw0)]),
        jnp.stack([begin1.astype(jnp.int32), jnp.int32(NWROW2), _pw(nw1)]),
    ])

    xp = jnp.pad(x, ((0, NPAD - N), (0, 0)))
    w3p = jnp.pad(W3, ((0, 0), (0, 88)))
    b3p = jnp.pad(b3, (0, 88)).reshape(1, 128)

    zeros_n = jnp.zeros((NPAD,), jnp.float32)
    ones_w = jnp.ones((WIN,), jnp.float32)

    part = _deg_kernel(dst_flat, zeros_n, ones_w)
    deg = part[0] + part[1] + 1.0
    dis = jnp.where(jnp.arange(NPAD) < N, deg ** -0.5, 0.0).reshape(NPAD, 1)

    h0 = _tc_l0(xp, W0, dis)
    a0 = _agg256(h0.reshape(NPAD, 2, 128), sdp, par).reshape(NPAD, 256)
    h1 = _tc_mid(a0, dis, b0.reshape(1, 256), W1, False)
    a1 = _agg256(h1.reshape(NPAD, 2, 128), sdp, par).reshape(NPAD, 256)
    h2 = _tc_mid(a1, dis, b1.reshape(1, 256), W2, False)
    a2 = _agg256(h2.reshape(NPAD, 2, 128), sdp, par).reshape(NPAD, 256)
    h3 = _tc_mid(a2, dis, b2.reshape(1, 256), w3p, True)
    a3 = _agg_esplit(h3, sd)
    out = _tc_final(a3, dis, b3p)
    return out[:N, :40]


# speedup vs baseline: 3.1161x; 3.1161x over previous
import functools

import jax
import jax.numpy as jnp
from jax import lax
from jax.experimental import pallas as pl
from jax.experimental.pallas import tpu as pltpu
from jax.experimental.pallas import tpu_sc as plsc

N = 10000
NPAD = 10240
E = 320000
EPAD = 327680
WIN = 128
NSUB = 16
NCORE = 2
ROWS_PER_SUB = NPAD // NSUB

_mesh = plsc.VectorSubcoreMesh(core_axis_name="c", subcore_axis_name="s")


def _deg_body(dst_hbm, zeros_hbm, ones_hbm, out_hbm, deg_sp, idx_v, ones_v, sem):
    c = lax.axis_index("c")
    s = lax.axis_index("s")
    pltpu.sync_copy(zeros_hbm.at[pl.ds(s * ROWS_PER_SUB, ROWS_PER_SUB)],
                    deg_sp.at[pl.ds(s * ROWS_PER_SUB, ROWS_PER_SUB)])
    pltpu.sync_copy(ones_hbm, ones_v)
    plsc.subcore_barrier()
    per_worker = EPAD // (NCORE * NSUB)
    base = (c * NSUB + s) * per_worker

    def win(j, _):
        pltpu.sync_copy(dst_hbm.at[pl.ds(base + j * WIN, WIN)], idx_v)
        pltpu.sync_copy(ones_v, deg_sp.at[idx_v], add=True)
        return 0

    lax.fori_loop(0, per_worker // WIN, win, 0)
    plsc.subcore_barrier()
    pltpu.sync_copy(deg_sp.at[pl.ds(s * ROWS_PER_SUB, ROWS_PER_SUB)],
                    out_hbm.at[c].at[pl.ds(s * ROWS_PER_SUB, ROWS_PER_SUB)])


_deg_kernel = pl.kernel(
    _deg_body,
    out_type=jax.ShapeDtypeStruct((NCORE, NPAD), jnp.float32),
    mesh=_mesh,
    scratch_types=[
        pltpu.VMEM_SHARED((NPAD,), jnp.float32),
        pltpu.VMEM((WIN,), jnp.int32),
        pltpu.VMEM((WIN,), jnp.float32),
        pltpu.SemaphoreType.DMA,
    ],
)


NBUF = 2
NWROW = EPAD // WIN


def _agg_pipeline(gather_ref, acc_sp, src1d, dst1d, srcidx, dstidx, rows,
                  gsem, ssem, row_base, nwin):
    def load_fire(w0, b):
        off = (w0 + b) * WIN
        pltpu.sync_copy(src1d.at[pl.ds(off, WIN)], srcidx[b])
        pltpu.sync_copy(dst1d.at[pl.ds(off, WIN)], dstidx[b])
        return pltpu.async_copy(gather_ref.at[srcidx[b]], rows[b], gsem[b])

    gd = [load_fire(row_base, b) for b in range(NBUF)]
    for b in range(NBUF):
        gd[b].wait()
        pltpu.async_copy(rows[b], acc_sp.at[dstidx[b]], ssem[b], add=True)

    def macro(j, _):
        w0 = row_base + j * NBUF
        gd2 = []
        for b in range(NBUF):
            pltpu.make_async_copy(rows[b], acc_sp.at[dstidx[b]],
                                  ssem[b]).wait()
            gd2.append(load_fire(w0, b))
        for b in range(NBUF):
            gd2[b].wait()
            pltpu.async_copy(rows[b], acc_sp.at[dstidx[b]], ssem[b], add=True)
        return 0

    lax.fori_loop(1, nwin // NBUF, macro, 0)
    for b in range(NBUF):
        pltpu.make_async_copy(rows[b], acc_sp.at[dstidx[b]], ssem[b]).wait()


def _agg_body(hs_hbm, src1d_hbm, dst1d_hbm, out_hbm, acc_sp,
              si0, si1, di0, di1, r0, r1, g0, g1, s0, s1, *, esplit):
    c = lax.axis_index("c")
    s = lax.axis_index("s")
    pltpu.sync_copy(hs_hbm.at[c].at[pl.ds(s * ROWS_PER_SUB, ROWS_PER_SUB)],
                    acc_sp.at[pl.ds(s * ROWS_PER_SUB, ROWS_PER_SUB)])
    plsc.subcore_barrier()
    if esplit:
        w = c * NSUB + s
        row_base = w * (NWROW // (NCORE * NSUB))
        nwin = NWROW // (NCORE * NSUB)
        gref = hs_hbm.at[0]
    else:
        row_base = s * (NWROW // NSUB)
        nwin = NWROW // NSUB
        gref = hs_hbm.at[c]
    _agg_pipeline(gref, acc_sp, src1d_hbm, dst1d_hbm,
                  (si0, si1), (di0, di1),
                  (r0, r1), (g0, g1), (s0, s1),
                  row_base, nwin)
    plsc.subcore_barrier()
    pltpu.sync_copy(acc_sp.at[pl.ds(s * ROWS_PER_SUB, ROWS_PER_SUB)],
                    out_hbm.at[c].at[pl.ds(s * ROWS_PER_SUB, ROWS_PER_SUB)])


def _make_agg(esplit):
    return pl.kernel(
        functools.partial(_agg_body, esplit=esplit),
        out_type=jax.ShapeDtypeStruct((NCORE, NPAD, 128), jnp.float32),
        mesh=_mesh,
        scratch_types=[
            pltpu.VMEM_SHARED((NPAD, 128), jnp.float32),
            pltpu.VMEM((WIN,), jnp.int32),
            pltpu.VMEM((WIN,), jnp.int32),
            pltpu.VMEM((WIN,), jnp.int32),
            pltpu.VMEM((WIN,), jnp.int32),
            pltpu.VMEM((WIN, 128), jnp.float32),
            pltpu.VMEM((WIN, 128), jnp.float32),
            pltpu.SemaphoreType.DMA,
            pltpu.SemaphoreType.DMA,
            pltpu.SemaphoreType.DMA,
            pltpu.SemaphoreType.DMA,
        ],
    )


_agg128 = _make_agg(False)
_agg_esplit = _make_agg(True)


BM = 2048


def _l0_body(x_ref, w_ref, dis_ref, out_ref):
    h = jnp.dot(x_ref[...], w_ref[...], preferred_element_type=jnp.float32)
    h = h * dis_ref[...]
    out_ref[0] = h[:, :128]
    out_ref[1] = h[:, 128:]


def _mid_body(agg_ref, dis_ref, b_ref, w_ref, out_ref, *, fsplit):
    t = jnp.concatenate([agg_ref[0], agg_ref[1]], axis=1)
    y = jnp.maximum(t * dis_ref[...] + b_ref[...], 0.0)
    h = jnp.dot(y, w_ref[...], preferred_element_type=jnp.float32)
    h = h * dis_ref[...]
    if fsplit:
        out_ref[0] = h[:, :128]
        out_ref[1] = h[:, 128:]
    else:
        out_ref[0] = h
        out_ref[1] = jnp.zeros_like(h)


def _final_body(agg_ref, dis_ref, b_ref, out_ref):
    t = agg_ref[0] + agg_ref[1]
    y = t * dis_ref[...] + b_ref[...]
    col = lax.broadcasted_iota(jnp.int32, y.shape, 1)
    ym = jnp.where(col < 40, y, -jnp.inf)
    m = jnp.max(ym, axis=1, keepdims=True)
    lse = jnp.log(jnp.sum(jnp.exp(ym - m), axis=1, keepdims=True)) + m
    out_ref[...] = y - lse


def _tc_l0(x, w0, dis):
    grid = (NPAD // BM,)
    return pl.pallas_call(
        _l0_body,
        grid=grid,
        in_specs=[
            pl.BlockSpec((BM, 128), lambda i: (i, 0)),
            pl.BlockSpec((128, 256), lambda i: (0, 0)),
            pl.BlockSpec((BM, 1), lambda i: (i, 0)),
        ],
        out_specs=pl.BlockSpec((NCORE, BM, 128), lambda i: (0, i, 0)),
        out_shape=jax.ShapeDtypeStruct((NCORE, NPAD, 128), jnp.float32),
    )(x, w0, dis)


def _tc_mid(agg, dis, b, w, fsplit):
    grid = (NPAD // BM,)
    kout = w.shape[1]
    return pl.pallas_call(
        functools.partial(_mid_body, fsplit=fsplit),
        grid=grid,
        in_specs=[
            pl.BlockSpec((NCORE, BM, 128), lambda i: (0, i, 0)),
            pl.BlockSpec((BM, 1), lambda i: (i, 0)),
            pl.BlockSpec((1, 256), lambda i: (0, 0)),
            pl.BlockSpec((256, kout), lambda i: (0, 0)),
        ],
        out_specs=pl.BlockSpec((NCORE, BM, 128), lambda i: (0, i, 0)),
        out_shape=jax.ShapeDtypeStruct((NCORE, NPAD, 128), jnp.float32),
    )(agg, dis, b, w)


def _tc_final(agg, dis, b):
    grid = (NPAD // BM,)
    return pl.pallas_call(
        _final_body,
        grid=grid,
        in_specs=[
            pl.BlockSpec((NCORE, BM, 128), lambda i: (0, i, 0)),
            pl.BlockSpec((BM, 1), lambda i: (i, 0)),
            pl.BlockSpec((1, 128), lambda i: (0, 0)),
        ],
        out_specs=pl.BlockSpec((BM, 128), lambda i: (i, 0)),
        out_shape=jax.ShapeDtypeStruct((NPAD, 128), jnp.float32),
    )(agg, dis, b)


def kernel(x, edge_index, W0, b0, W1, b1, W2, b2, W3, b3):
    src = edge_index[0].astype(jnp.int32)
    dst = edge_index[1].astype(jnp.int32)
    pad = N + (jnp.arange(EPAD - E, dtype=jnp.int32) % (NPAD - N))
    src_p = jnp.concatenate([src, pad])
    dst_p = jnp.concatenate([dst, pad])
    dst_flat = dst_p

    xp = jnp.pad(x, ((0, NPAD - N), (0, 0)))
    w3p = jnp.pad(W3, ((0, 0), (0, 88)))
    b3p = jnp.pad(b3, (0, 88)).reshape(1, 128)

    zeros_n = jnp.zeros((NPAD,), jnp.float32)
    ones_w = jnp.ones((WIN,), jnp.float32)

    part = _deg_kernel(dst_flat, zeros_n, ones_w)
    deg = part[0] + part[1] + 1.0
    dis = (deg ** -0.5).reshape(NPAD, 1)

    h0 = _tc_l0(xp, W0, dis)
    a0 = _agg128(h0, src_p, dst_p)
    h1 = _tc_mid(a0, dis, b0.reshape(1, 256), W1, True)
    a1 = _agg128(h1, src_p, dst_p)
    h2 = _tc_mid(a1, dis, b1.reshape(1, 256), W2, True)
    a2 = _agg128(h2, src_p, dst_p)
    h3 = _tc_mid(a2, dis, b2.reshape(1, 256), w3p, False)
    a3 = _agg_esplit(h3, src_p, dst_p)
    out = _tc_final(a3, dis, b3p)
    return out[:N, :40]
